# baseline (speedup 1.0000x reference)
import jax
import jax.numpy as jnp
from jax import lax
from jax.experimental import pallas as pl
from jax.experimental.pallas import tpu as pltpu
from jax.experimental.pallas import tpu_sc as plsc

NUM_SEG = 512
NUM_NODES = 100000
FEAT = 128
CHUNK = 128
NW = 32
PER_W = 25
ROWS_W = PER_W * CHUNK
LAST_FULL = 6
TAIL_BASE = NW * ROWS_W - ROWS_W + LAST_FULL * CHUNK
TAIL = NUM_NODES - TAIL_BASE
NBUF = 4
RSTEPS = (PER_W + NBUF - 1) // NBUF
SEG_PER_TILE = NUM_SEG // 16
IDS_ROWS = 32


def _seg_body(attr_hbm, batch_hbm, b4d_hbm, psum_hbm, pcnt_hbm,
              chunks, ids2d, tail_ids, ones_v, stage_v, cstage_v,
              acc_sh, cnt_sh, sgs, ssd, scnt):
    cid = lax.axis_index("c")
    sid = lax.axis_index("s")
    w = cid * 16 + sid

    zeros16 = jnp.zeros((16,), jnp.float32)
    ones16 = jnp.ones((16,), jnp.float32)
    for i in range(SEG_PER_TILE):
        for j in range(FEAT // 16):
            stage_v[i, pl.ds(j * 16, 16)] = zeros16
    for i in range(SEG_PER_TILE // 16):
        cstage_v[pl.ds(i * 16, 16)] = zeros16
    for i in range(CHUNK // 16):
        ones_v[pl.ds(i * 16, 16)] = ones16

    pltpu.sync_copy(stage_v, acc_sh.at[pl.ds(sid * SEG_PER_TILE, SEG_PER_TILE)])
    pltpu.sync_copy(cstage_v, cnt_sh.at[pl.ds(sid * SEG_PER_TILE, SEG_PER_TILE)])
    plsc.subcore_barrier()

    last = w == NW - 1
    n_c = jnp.where(last, LAST_FULL, PER_W)
    row0 = ROWS_W * w

    pltpu.sync_copy(b4d_hbm.at[pl.ds(IDS_ROWS * w, IDS_ROWS)], ids2d)

    def gather(i, b):
        pltpu.async_copy(attr_hbm.at[pl.ds(row0 + i * CHUNK, CHUNK)],
                         chunks[b], sgs[b])

    def gather_wait(b):
        pltpu.make_async_copy(attr_hbm.at[pl.ds(0, CHUNK)], chunks[b], sgs[b]).wait()

    def scatter(i, b):
        pltpu.async_copy(chunks[b], acc_sh.at[ids2d.at[i]], ssd[b], add=True)
        pltpu.async_copy(ones_v, cnt_sh.at[ids2d.at[i]], scnt, add=True)

    def scatter_wait(b):
        pltpu.make_async_copy(chunks[b], acc_sh.at[pl.ds(0, CHUNK)], ssd[b]).wait()

    for b in range(NBUF):
        gather(b, b)

    for t in range(RSTEPS):
        for b in range(NBUF):
            i = NBUF * t + b
            if i >= PER_W:
                continue

            @pl.when(i < n_c)
            def _():
                gather_wait(b)
                scatter(i, b)

        for b in range(NBUF):
            i = NBUF * t + b
            if i >= PER_W:
                continue

            @pl.when(i + NBUF < n_c)
            def _():
                scatter_wait(b)
                gather(i + NBUF, b)

    for b in range(NBUF):
        scatter_wait(b)

    @pl.when(last)
    def _():
        pltpu.sync_copy(attr_hbm.at[pl.ds(TAIL_BASE, TAIL)],
                        chunks[0].at[pl.ds(0, TAIL)])
        pltpu.sync_copy(batch_hbm.at[pl.ds(TAIL_BASE, TAIL)], tail_ids)
        pltpu.sync_copy(chunks[0].at[pl.ds(0, TAIL)], acc_sh.at[tail_ids], add=True)
        pltpu.sync_copy(ones_v.at[pl.ds(0, TAIL)], cnt_sh.at[tail_ids], add=True)

    for i in range(PER_W):
        @pl.when(i < n_c)
        def _():
            pltpu.make_async_copy(ones_v, cnt_sh.at[pl.ds(0, CHUNK)], scnt).wait()

    plsc.subcore_barrier()

    row = sid * SEG_PER_TILE
    pltpu.sync_copy(acc_sh.at[pl.ds(row, SEG_PER_TILE)], stage_v)
    pltpu.sync_copy(stage_v, psum_hbm.at[pl.ds(cid * NUM_SEG + row, SEG_PER_TILE)])
    pltpu.sync_copy(cnt_sh.at[pl.ds(row, SEG_PER_TILE)], cstage_v)
    pltpu.sync_copy(cstage_v, pcnt_hbm.at[cid, pl.ds(row, SEG_PER_TILE)])


def _body_wrapper(attr_hbm, batch_hbm, b4d_hbm, psum_hbm, pcnt_hbm,
                  c0, c1, c2, c3, ids2d, tail_ids, ones_v, stage_v, cstage_v,
                  acc_sh, cnt_sh, g0, g1, g2, g3, s0, s1, s2, s3, scnt):
    _seg_body(attr_hbm, batch_hbm, b4d_hbm, psum_hbm, pcnt_hbm,
              [c0, c1, c2, c3], ids2d, tail_ids, ones_v, stage_v, cstage_v,
              acc_sh, cnt_sh, [g0, g1, g2, g3], [s0, s1, s2, s3], scnt)


_seg_kernel = pl.kernel(
    _body_wrapper,
    out_type=[
        jax.ShapeDtypeStruct((2 * NUM_SEG, FEAT), jnp.float32),
        jax.ShapeDtypeStruct((16, NUM_SEG), jnp.float32),
    ],
    mesh=plsc.VectorSubcoreMesh(core_axis_name="c", subcore_axis_name="s"),
    scratch_types=(
        [pltpu.VMEM((CHUNK, FEAT), jnp.float32)] * NBUF
        + [
            pltpu.VMEM((IDS_ROWS, CHUNK), jnp.int32),
            pltpu.VMEM((TAIL,), jnp.int32),
            pltpu.VMEM((CHUNK,), jnp.float32),
            pltpu.VMEM((SEG_PER_TILE, FEAT), jnp.float32),
            pltpu.VMEM((SEG_PER_TILE,), jnp.float32),
            pltpu.VMEM_SHARED((NUM_SEG, FEAT), jnp.float32),
            pltpu.VMEM_SHARED((NUM_SEG,), jnp.float32),
        ]
        + [pltpu.SemaphoreType.DMA] * (2 * NBUF + 1)
    ),
)


def _combine_body(ps_ref, pc_ref, o_ref):
    s = ps_ref[0:NUM_SEG, :] + ps_ref[NUM_SEG:2 * NUM_SEG, :]
    ct = jnp.transpose(pc_ref[...], (1, 0))
    c = ct[:, 0:1] + ct[:, 1:2]
    o_ref[...] = s / jnp.maximum(c, 1.0)


def kernel(node_attr, batch):
    bpad = jnp.pad(batch, (0, NW * ROWS_W - NUM_NODES))
    b4d = jnp.pad(bpad.reshape(NW, PER_W, CHUNK),
                  ((0, 0), (0, IDS_ROWS - PER_W), (0, 0))).reshape(-1, CHUNK)
    psum, pcnt = _seg_kernel(node_attr, batch, b4d)
    mean = pl.pallas_call(
        _combine_body,
        out_shape=jax.ShapeDtypeStruct((NUM_SEG, FEAT), jnp.float32),
    )(psum, pcnt)
    return mean.reshape(-1)
